# baseline (speedup 1.0000x reference)
import functools

import jax
import jax.numpy as jnp
from jax import lax
from jax.experimental import pallas as pl
from jax.experimental.pallas import tpu as pltpu
from jax.experimental.pallas import tpu_sc as plsc

NC, NS = 2, 16
NW = NC * NS
IDXW = 128
R = 4
CHUNK = R * IDXW
C2 = CHUNK // 2


def _sc_gather(ids2d, weight):
    n_rows, _ = ids2d.shape
    B = n_rows * IDXW
    V, D = weight.shape
    rows_per_w = n_rows // NW
    n_chunks = rows_per_w // R
    p_per_w = rows_per_w * IDXW // 2

    mesh = plsc.VectorSubcoreMesh(core_axis_name="c", subcore_axis_name="s")

    @functools.partial(
        pl.kernel,
        out_type=jax.ShapeDtypeStruct((B // 2, 2 * D), jnp.float32),
        mesh=mesh,
        compiler_params=pltpu.CompilerParams(use_tc_tiling_on_sc=False),
        scratch_types=[
            pltpu.VMEM((R, IDXW), jnp.int32),
            pltpu.VMEM((CHUNK, D), jnp.float32),
            pltpu.SemaphoreType.DMA,
        ],
    )
    def k(ids_hbm, w_hbm, out_hbm, idx_v, rows_v, sem):
        wid = lax.axis_index("s") * NC + lax.axis_index("c")
        row_base = wid * rows_per_w
        out_base = wid * p_per_w

        @pl.loop(0, n_chunks)
        def _(j):
            row = row_base + j * R
            pltpu.sync_copy(ids_hbm.at[pl.ds(row, R)], idx_v)
            copies = [
                pltpu.async_copy(
                    w_hbm.at[idx_v.at[r]],
                    rows_v.at[pl.ds(r * IDXW, IDXW)],
                    sem,
                )
                for r in range(R)
            ]
            for c in copies:
                c.wait()
            oj = out_base + j * C2
            pltpu.sync_copy(rows_v.at[pl.ds(0, C2)],
                            out_hbm.at[pl.ds(oj, C2), pl.ds(0, D)])
            pltpu.sync_copy(rows_v.at[pl.ds(C2, C2)],
                            out_hbm.at[pl.ds(oj, C2), pl.ds(D, D)])

    return k(ids2d, weight)


def _tc_unpack_transpose(packed3, Bt, L, D, lb):
    gpl = Bt // 512
    x4 = packed3.reshape(L, gpl, 256, 2 * D)

    def body(x_ref, o_ref):
        x = x_ref[...].reshape(lb, 256, 2 * D)
        xt = jnp.transpose(x, (2, 0, 1))
        y = xt.reshape(2, D, lb, 256)
        y = jnp.transpose(y, (1, 2, 0, 3))
        o_ref[...] = y.reshape(D, lb, 512)

    return pl.pallas_call(
        body,
        grid=(gpl, L // lb),
        in_specs=[pl.BlockSpec((lb, 1, 256, 2 * D), lambda g, j: (j, g, 0, 0))],
        out_specs=pl.BlockSpec((D, lb, 512), lambda g, j: (0, j, g)),
        out_shape=jax.ShapeDtypeStruct((D, L, Bt), jnp.float32),
    )(x4)


def kernel(text_ids, weight):
    Bt, L = text_ids.shape
    V, D = weight.shape
    ids_lmajor = text_ids.T.reshape(L * Bt // IDXW, IDXW).astype(jnp.int32)
    packed = _sc_gather(ids_lmajor, weight)
    def _copy_body(x_ref, o_ref):
        o_ref[...] = x_ref[...]

    copied = pl.pallas_call(
        _copy_body,
        grid=(50,),
        in_specs=[pl.BlockSpec((8192, 128), lambda i: (i, 0))],
        out_specs=pl.BlockSpec((8192, 128), lambda i: (i, 0)),
        out_shape=jax.ShapeDtypeStruct((L * Bt // 2, 2 * D), jnp.float32),
    )(packed)
    return jnp.transpose(copied.reshape(D, L, Bt), (2, 0, 1))


# speedup vs baseline: 1.0607x; 1.0607x over previous
import functools

import jax
import jax.numpy as jnp
from jax import lax
from jax.experimental import pallas as pl
from jax.experimental.pallas import tpu as pltpu
from jax.experimental.pallas import tpu_sc as plsc

NC, NS = 2, 16
NW = NC * NS
IDXW = 128
R = 4
CHUNK = R * IDXW
C2 = CHUNK // 2


def _sc_gather(ids2d, weight):
    n_rows, _ = ids2d.shape
    B = n_rows * IDXW
    V, D = weight.shape
    rows_per_w = n_rows // NW
    n_chunks = rows_per_w // R
    p_per_w = rows_per_w * IDXW // 2

    mesh = plsc.VectorSubcoreMesh(core_axis_name="c", subcore_axis_name="s")

    @functools.partial(
        pl.kernel,
        out_type=jax.ShapeDtypeStruct((B // 2, 2 * D), jnp.float32),
        mesh=mesh,
        compiler_params=pltpu.CompilerParams(use_tc_tiling_on_sc=False),
        scratch_types=[
            pltpu.VMEM((R, IDXW), jnp.int32),
            pltpu.VMEM((CHUNK, D), jnp.float32),
            pltpu.SemaphoreType.DMA,
        ],
    )
    def k(ids_hbm, w_hbm, out_hbm, idx_v, rows_v, sem):
        wid = lax.axis_index("s") * NC + lax.axis_index("c")
        row_base = wid * rows_per_w
        out_base = wid * p_per_w

        @pl.loop(0, n_chunks)
        def _(j):
            row = row_base + j * R
            pltpu.sync_copy(ids_hbm.at[pl.ds(row, R)], idx_v)
            copies = [
                pltpu.async_copy(
                    w_hbm.at[idx_v.at[r]],
                    rows_v.at[pl.ds(r * IDXW, IDXW)],
                    sem,
                )
                for r in range(R)
            ]
            for c in copies:
                c.wait()
            oj = out_base + j * C2
            pltpu.sync_copy(rows_v.at[pl.ds(0, C2)],
                            out_hbm.at[pl.ds(oj, C2), pl.ds(0, D)])
            pltpu.sync_copy(rows_v.at[pl.ds(C2, C2)],
                            out_hbm.at[pl.ds(oj, C2), pl.ds(D, D)])

    return k(ids2d, weight)


def _tc_unpack_transpose(packed3, Bt, L, D, lb):
    gpl = Bt // 512
    x4 = packed3.reshape(L, gpl, 256, 2 * D)

    def body(x_ref, o_ref):
        x = x_ref[...].reshape(lb, 256, 2 * D)
        xt = jnp.transpose(x, (2, 0, 1))
        y = xt.reshape(2, D, lb, 256)
        y = jnp.transpose(y, (1, 2, 0, 3))
        o_ref[...] = y.reshape(D, lb, 512)

    return pl.pallas_call(
        body,
        grid=(gpl, L // lb),
        in_specs=[pl.BlockSpec((lb, 1, 256, 2 * D), lambda g, j: (j, g, 0, 0))],
        out_specs=pl.BlockSpec((D, lb, 512), lambda g, j: (0, j, g)),
        out_shape=jax.ShapeDtypeStruct((D, L, Bt), jnp.float32),
        compiler_params=pltpu.CompilerParams(
            dimension_semantics=("parallel", "parallel")),
    )(x4)


def kernel(text_ids, weight):
    Bt, L = text_ids.shape
    V, D = weight.shape
    ids_lmajor = text_ids.T.reshape(L * Bt // IDXW, IDXW).astype(jnp.int32)
    packed = _sc_gather(ids_lmajor, weight)
    packed3 = packed.reshape(L * Bt // 512, 256, 2 * D)
    out_t = _tc_unpack_transpose(packed3, Bt, L, D, 8)
    return jnp.transpose(out_t, (2, 0, 1))


# speedup vs baseline: 1.1410x; 1.0757x over previous
import functools

import jax
import jax.numpy as jnp
from jax import lax
from jax.experimental import pallas as pl
from jax.experimental.pallas import tpu as pltpu
from jax.experimental.pallas import tpu_sc as plsc

NC, NS = 2, 16
NW = NC * NS
IDXW = 128
R = 4
CHUNK = R * IDXW
C2 = CHUNK // 2


def _sc_gather(ids2d, weight):
    n_rows, _ = ids2d.shape
    B = n_rows * IDXW
    V, D = weight.shape
    rows_per_w = n_rows // NW
    n_chunks = rows_per_w // R
    p_per_w = rows_per_w * IDXW // 2

    mesh = plsc.VectorSubcoreMesh(core_axis_name="c", subcore_axis_name="s")

    @functools.partial(
        pl.kernel,
        out_type=jax.ShapeDtypeStruct((B // 2, 2 * D), jnp.float32),
        mesh=mesh,
        compiler_params=pltpu.CompilerParams(use_tc_tiling_on_sc=False),
        scratch_types=[
            pltpu.VMEM((R, IDXW), jnp.int32),
            pltpu.VMEM((CHUNK, D), jnp.float32),
            pltpu.SemaphoreType.DMA,
        ],
    )
    def k(ids_hbm, w_hbm, out_hbm, idx_v, rows_v, sem):
        wid = lax.axis_index("s") * NC + lax.axis_index("c")
        row_base = wid * rows_per_w
        out_base = wid * p_per_w

        @pl.loop(0, n_chunks)
        def _(j):
            row = row_base + j * R
            pltpu.sync_copy(ids_hbm.at[pl.ds(row, R)], idx_v)
            copies = [
                pltpu.async_copy(
                    w_hbm.at[idx_v.at[r]],
                    rows_v.at[pl.ds(r * IDXW, IDXW)],
                    sem,
                )
                for r in range(R)
            ]
            for c in copies:
                c.wait()
            oj = out_base + j * C2
            pltpu.sync_copy(rows_v.at[pl.ds(0, C2)],
                            out_hbm.at[pl.ds(oj, C2), pl.ds(0, D)])
            pltpu.sync_copy(rows_v.at[pl.ds(C2, C2)],
                            out_hbm.at[pl.ds(oj, C2), pl.ds(D, D)])

    return k(ids2d, weight)


def _tc_unpack_transpose(packed3, Bt, L, D, lb):
    gpl = Bt // 512

    def body(x_ref, o_ref):
        x = x_ref[...]
        xt = jnp.transpose(x, (2, 0, 1))
        y = xt.reshape(2, D, lb, gpl, 256)
        y = jnp.transpose(y, (1, 2, 3, 0, 4))
        o_ref[...] = y.reshape(D, lb, Bt)

    return pl.pallas_call(
        body,
        grid=(L // lb,),
        in_specs=[pl.BlockSpec((lb * gpl, 256, 2 * D), lambda i: (i, 0, 0))],
        out_specs=pl.BlockSpec((D, lb, Bt), lambda i: (0, i, 0)),
        out_shape=jax.ShapeDtypeStruct((D, L, Bt), jnp.float32),
        compiler_params=pltpu.CompilerParams(
            dimension_semantics=("parallel",)),
    )(packed3)


def kernel(text_ids, weight):
    Bt, L = text_ids.shape
    V, D = weight.shape
    ids_lmajor = text_ids.T.reshape(L * Bt // IDXW, IDXW).astype(jnp.int32)
    packed = _sc_gather(ids_lmajor, weight)
    packed3 = packed.reshape(L * Bt // 512, 256, 2 * D)
    out_t = _tc_unpack_transpose(packed3, Bt, L, D, 8)
    return jnp.transpose(out_t, (2, 0, 1))
